# baseline (speedup 1.0000x reference)
import functools

import jax
import jax.numpy as jnp
from jax import lax
from jax.experimental import pallas as pl
from jax.experimental.pallas import tpu as pltpu
from jax.experimental.pallas import tpu_sc as plsc

N_EDGES = 3_200_000
N_ROWS = 100_000

NC = 2
NS = 16
NW = NC * NS
LANES = 128
EROWS = N_EDGES // LANES
HR = 800
HC = 128
K = 32
CE = K * LANES
FULL = 768
EXTRA = 16
TAIL = 8
NPAIRS = (FULL // K) // 2

_mesh = plsc.VectorSubcoreMesh(
    core_axis_name="c", subcore_axis_name="s", num_cores=NC, num_subcores=NS
)


@functools.partial(
    pl.kernel,
    out_type=jax.ShapeDtypeStruct((NW, HR, HC), jnp.float32),
    mesh=_mesh,
    compiler_params=pltpu.CompilerParams(needs_layout_passes=False),
    scratch_types=[
        pltpu.VMEM((HR, HC), jnp.float32),
        pltpu.VMEM((2, CE), jnp.int32),
        pltpu.VMEM((2, CE), jnp.float32),
        pltpu.SemaphoreType.DMA,
        pltpu.SemaphoreType.DMA,
    ],
)
def _segment_sum_sc(ei_hbm, w_hbm, hist_hbm, hist, idx_buf, w_buf, ld0, ld1):
    c = lax.axis_index("c")
    s = lax.axis_index("s")
    tid = c * NS + s
    ld = (ld0, ld1)

    base = tid * 776 + 8 * jnp.minimum(tid, 21)

    def _load(e0, b, n):
        pltpu.async_copy(
            ei_hbm.at[1, pl.ds(e0 * LANES, n * LANES)],
            idx_buf.at[b, pl.ds(0, n * LANES)], ld[b],
        )
        pltpu.async_copy(
            w_hbm.at[pl.ds(e0 * LANES, n * LANES)],
            w_buf.at[b, pl.ds(0, n * LANES)], ld[b],
        )

    def _drain_load(b, n):
        pltpu.make_async_copy(
            ei_hbm.at[1, pl.ds(0, n * LANES)],
            idx_buf.at[b, pl.ds(0, n * LANES)], ld[b],
        ).wait()
        pltpu.make_async_copy(
            w_hbm.at[pl.ds(0, n * LANES)],
            w_buf.at[b, pl.ds(0, n * LANES)], ld[b],
        ).wait()

    def _scatter_rows(b, nrows):
        def _grp(i):
            sl = pl.ds(i * 16, 16)
            idx16 = idx_buf[b, sl]
            hi = lax.shift_right_logical(idx16, 7)
            lo = jnp.bitwise_and(idx16, 127)
            plsc.addupdate_scatter(hist, [hi, lo], w_buf[b, sl])

        plsc.parallel_loop(0, nrows * (LANES // 16), unroll=32)(_grp)

    _load(base, 0, K)

    z16 = jnp.zeros((16,), jnp.float32)

    def _zero(i):
        hist[lax.shift_right_logical(i, 3),
             pl.ds(jnp.bitwise_and(i, 7) * 16, 16)] = z16

    plsc.parallel_loop(0, HR * (HC // 16), unroll=8)(_zero)

    def _pair(gg, _):
        for b in range(2):
            g = gg * 2 + b

            @pl.when(g + 1 < FULL // K)
            def _():
                _load(base + (g + 1) * K, 1 - b, K)

            _drain_load(b, K)
            _scatter_rows(b, K)
        return 0

    lax.fori_loop(0, NPAIRS, _pair, 0)

    @pl.when(tid < 21)
    def _():
        _load(base + FULL, 0, EXTRA)
        _drain_load(0, EXTRA)
        _scatter_rows(0, EXTRA)

    @pl.when(tid >= 21)
    def _():
        _load(base + FULL, 1, TAIL)
        _drain_load(1, TAIL)
        _scatter_rows(1, TAIL)

    pltpu.sync_copy(hist, hist_hbm.at[tid])


def _finalize_tc(p_ref, inv_ref, rs_ref):
    p = jnp.sum(p_ref[...], axis=0)
    inv = 1.0 / (p + 1e-8)
    inv_ref[...] = inv
    rs_ref[...] = p * inv


_finalize = pl.pallas_call(
    _finalize_tc,
    out_shape=[
        jax.ShapeDtypeStruct((HR, HC), jnp.float32),
        jax.ShapeDtypeStruct((HR, HC), jnp.float32),
    ],
)


@functools.partial(
    pl.kernel,
    out_type=jax.ShapeDtypeStruct((N_EDGES,), jnp.float32),
    mesh=_mesh,
    compiler_params=pltpu.CompilerParams(needs_layout_passes=False),
    scratch_types=[
        pltpu.VMEM((HR, HC), jnp.float32),
        pltpu.VMEM((2, CE), jnp.int32),
        pltpu.VMEM((2, CE), jnp.float32),
        pltpu.VMEM((2, CE), jnp.float32),
        pltpu.SemaphoreType.DMA,
        pltpu.SemaphoreType.DMA,
        pltpu.SemaphoreType.DMA,
        pltpu.SemaphoreType.DMA,
    ],
)
def _gather_mul_sc(
    ei_hbm, w_hbm, inv_hbm, wn_hbm,
    inv_vmem, idx_buf, w_buf, out_buf, ld0, ld1, st0, st1,
):
    c = lax.axis_index("c")
    s = lax.axis_index("s")
    tid = c * NS + s
    ld = (ld0, ld1)
    st = (st0, st1)

    bcps = [
        pltpu.async_copy(
            inv_hbm.at[pl.ds(q * (HR // 4), HR // 4)],
            inv_vmem.at[pl.ds(q * (HR // 4), HR // 4)], st0,
        )
        for q in range(4)
    ]
    for bcp in bcps:
        bcp.wait()

    base = tid * 776 + 8 * jnp.minimum(tid, 21)

    def _load(e0, b, n):
        pltpu.async_copy(
            ei_hbm.at[1, pl.ds(e0 * LANES, n * LANES)],
            idx_buf.at[b, pl.ds(0, n * LANES)], ld[b],
        )
        pltpu.async_copy(
            w_hbm.at[pl.ds(e0 * LANES, n * LANES)],
            w_buf.at[b, pl.ds(0, n * LANES)], ld[b],
        )

    def _drain_load(b, n):
        pltpu.make_async_copy(
            ei_hbm.at[1, pl.ds(0, n * LANES)],
            idx_buf.at[b, pl.ds(0, n * LANES)], ld[b],
        ).wait()
        pltpu.make_async_copy(
            w_hbm.at[pl.ds(0, n * LANES)],
            w_buf.at[b, pl.ds(0, n * LANES)], ld[b],
        ).wait()

    def _gather_rows(b, nrows):
        def _grp(i):
            sl = pl.ds(i * 16, 16)
            idx16 = idx_buf[b, sl]
            hi = lax.shift_right_logical(idx16, 7)
            lo = jnp.bitwise_and(idx16, 127)
            g16 = plsc.load_gather(inv_vmem, [hi, lo])
            out_buf[b, sl] = w_buf[b, sl] * g16

        plsc.parallel_loop(0, nrows * (LANES // 16), unroll=32)(_grp)

    _load(base, 0, K)

    def _pair(gg, _):
        for b in range(2):
            g = gg * 2 + b

            @pl.when(g + 1 < FULL // K)
            def _():
                _load(base + (g + 1) * K, 1 - b, K)

            _drain_load(b, K)

            @pl.when(gg >= 1)
            def _():
                pltpu.make_async_copy(
                    out_buf.at[b], wn_hbm.at[pl.ds(0, CE)], st[b]
                ).wait()

            _gather_rows(b, K)
            pltpu.async_copy(
                out_buf.at[b], wn_hbm.at[pl.ds((base + g * K) * LANES, CE)], st[b]
            )
        return 0

    lax.fori_loop(0, NPAIRS, _pair, 0)
    pltpu.make_async_copy(out_buf.at[0], wn_hbm.at[pl.ds(0, CE)], st[0]).wait()
    pltpu.make_async_copy(out_buf.at[1], wn_hbm.at[pl.ds(0, CE)], st[1]).wait()

    @pl.when(tid < 21)
    def _():
        _load(base + FULL, 0, EXTRA)
        _drain_load(0, EXTRA)
        _gather_rows(0, EXTRA)
        pltpu.sync_copy(
            out_buf.at[0, pl.ds(0, EXTRA * LANES)],
            wn_hbm.at[pl.ds((base + FULL) * LANES, EXTRA * LANES)],
        )

    @pl.when(tid >= 21)
    def _():
        _load(base + FULL, 1, TAIL)
        _drain_load(1, TAIL)
        _gather_rows(1, TAIL)
        pltpu.sync_copy(
            out_buf.at[1, pl.ds(0, TAIL * LANES)],
            wn_hbm.at[pl.ds((base + FULL) * LANES, TAIL * LANES)],
        )


def kernel(edge_index, weights):
    partial = _segment_sum_sc(edge_index, weights)
    inv_norm, row_sums = _finalize(partial)
    wn = _gather_mul_sc(edge_index, weights, inv_norm)
    return wn, row_sums.reshape(-1)[:N_ROWS]


# speedup vs baseline: 1.0188x; 1.0188x over previous
import functools

import jax
import jax.numpy as jnp
from jax import lax
from jax.experimental import pallas as pl
from jax.experimental.pallas import tpu as pltpu
from jax.experimental.pallas import tpu_sc as plsc

N_EDGES = 3_200_000
N_ROWS = 100_000

NC = 2
NS = 16
NW = NC * NS
LANES = 128
EROWS = N_EDGES // LANES
HR = 800
HC = 128
K = 32
CE = K * LANES
FULL = 768
EXTRA = 16
TAIL = 8
NPAIRS = (FULL // K) // 2

_mesh = plsc.VectorSubcoreMesh(
    core_axis_name="c", subcore_axis_name="s", num_cores=NC, num_subcores=NS
)


@functools.partial(
    pl.kernel,
    out_type=jax.ShapeDtypeStruct((NW, HR, HC), jnp.float32),
    mesh=_mesh,
    compiler_params=pltpu.CompilerParams(needs_layout_passes=False),
    scratch_types=[
        pltpu.VMEM((HR, HC), jnp.float32),
        pltpu.VMEM((2, CE), jnp.int32),
        pltpu.VMEM((2, CE), jnp.float32),
        pltpu.SemaphoreType.DMA,
        pltpu.SemaphoreType.DMA,
    ],
)
def _segment_sum_sc(ei_hbm, w_hbm, hist_hbm, hist, idx_buf, w_buf, ld0, ld1):
    c = lax.axis_index("c")
    s = lax.axis_index("s")
    tid = c * NS + s
    ld = (ld0, ld1)

    base = tid * 776 + 8 * jnp.minimum(tid, 21)

    def _load(e0, b, n):
        pltpu.async_copy(
            ei_hbm.at[1, pl.ds(e0 * LANES, n * LANES)],
            idx_buf.at[b, pl.ds(0, n * LANES)], ld[b],
        )
        pltpu.async_copy(
            w_hbm.at[pl.ds(e0 * LANES, n * LANES)],
            w_buf.at[b, pl.ds(0, n * LANES)], ld[b],
        )

    def _drain_load(b, n):
        pltpu.make_async_copy(
            ei_hbm.at[1, pl.ds(0, n * LANES)],
            idx_buf.at[b, pl.ds(0, n * LANES)], ld[b],
        ).wait()
        pltpu.make_async_copy(
            w_hbm.at[pl.ds(0, n * LANES)],
            w_buf.at[b, pl.ds(0, n * LANES)], ld[b],
        ).wait()

    def _scatter_rows(b, nrows):
        def _grp(i):
            sl = pl.ds(i * 16, 16)
            idx16 = idx_buf[b, sl]
            hi = lax.shift_right_logical(idx16, 7)
            lo = jnp.bitwise_and(idx16, 127)
            plsc.addupdate_scatter(hist, [hi, lo], w_buf[b, sl])

        plsc.parallel_loop(0, nrows * (LANES // 16), unroll=16)(_grp)

    _load(base, 0, K)

    z16 = jnp.zeros((16,), jnp.float32)

    def _zero(i):
        hist[lax.shift_right_logical(i, 3),
             pl.ds(jnp.bitwise_and(i, 7) * 16, 16)] = z16

    plsc.parallel_loop(0, HR * (HC // 16), unroll=8)(_zero)

    def _pair(gg, _):
        for b in range(2):
            g = gg * 2 + b

            @pl.when(g + 1 < FULL // K)
            def _():
                _load(base + (g + 1) * K, 1 - b, K)

            _drain_load(b, K)
            _scatter_rows(b, K)
        return 0

    lax.fori_loop(0, NPAIRS, _pair, 0)

    @pl.when(tid < 21)
    def _():
        _load(base + FULL, 0, EXTRA)
        _drain_load(0, EXTRA)
        _scatter_rows(0, EXTRA)

    @pl.when(tid >= 21)
    def _():
        _load(base + FULL, 1, TAIL)
        _drain_load(1, TAIL)
        _scatter_rows(1, TAIL)

    pltpu.sync_copy(hist, hist_hbm.at[tid])


def _finalize_tc(p_ref, inv_ref, rs_ref):
    p = jnp.sum(p_ref[...], axis=0)
    inv = 1.0 / (p + 1e-8)
    inv_ref[...] = inv
    rs_ref[...] = p * inv


_finalize = pl.pallas_call(
    _finalize_tc,
    out_shape=[
        jax.ShapeDtypeStruct((HR, HC), jnp.float32),
        jax.ShapeDtypeStruct((HR, HC), jnp.float32),
    ],
)


@functools.partial(
    pl.kernel,
    out_type=jax.ShapeDtypeStruct((N_EDGES,), jnp.float32),
    mesh=_mesh,
    compiler_params=pltpu.CompilerParams(needs_layout_passes=False),
    scratch_types=[
        pltpu.VMEM((HR, HC), jnp.float32),
        pltpu.VMEM((2, CE), jnp.int32),
        pltpu.VMEM((2, CE), jnp.float32),
        pltpu.VMEM((2, CE), jnp.float32),
        pltpu.SemaphoreType.DMA,
        pltpu.SemaphoreType.DMA,
        pltpu.SemaphoreType.DMA,
        pltpu.SemaphoreType.DMA,
    ],
)
def _gather_mul_sc(
    ei_hbm, w_hbm, inv_hbm, wn_hbm,
    inv_vmem, idx_buf, w_buf, out_buf, ld0, ld1, st0, st1,
):
    c = lax.axis_index("c")
    s = lax.axis_index("s")
    tid = c * NS + s
    ld = (ld0, ld1)
    st = (st0, st1)

    bcps = [
        pltpu.async_copy(
            inv_hbm.at[pl.ds(q * (HR // 4), HR // 4)],
            inv_vmem.at[pl.ds(q * (HR // 4), HR // 4)], st0,
        )
        for q in range(4)
    ]
    base = tid * 776 + 8 * jnp.minimum(tid, 21)

    def _load(e0, b, n):
        pltpu.async_copy(
            ei_hbm.at[1, pl.ds(e0 * LANES, n * LANES)],
            idx_buf.at[b, pl.ds(0, n * LANES)], ld[b],
        )
        pltpu.async_copy(
            w_hbm.at[pl.ds(e0 * LANES, n * LANES)],
            w_buf.at[b, pl.ds(0, n * LANES)], ld[b],
        )

    def _drain_load(b, n):
        pltpu.make_async_copy(
            ei_hbm.at[1, pl.ds(0, n * LANES)],
            idx_buf.at[b, pl.ds(0, n * LANES)], ld[b],
        ).wait()
        pltpu.make_async_copy(
            w_hbm.at[pl.ds(0, n * LANES)],
            w_buf.at[b, pl.ds(0, n * LANES)], ld[b],
        ).wait()

    def _gather_rows(b, nrows):
        def _grp(i):
            sl = pl.ds(i * 16, 16)
            idx16 = idx_buf[b, sl]
            hi = lax.shift_right_logical(idx16, 7)
            lo = jnp.bitwise_and(idx16, 127)
            g16 = plsc.load_gather(inv_vmem, [hi, lo])
            out_buf[b, sl] = w_buf[b, sl] * g16

        plsc.parallel_loop(0, nrows * (LANES // 16), unroll=16)(_grp)

    _load(base, 0, K)
    for bcp in bcps:
        bcp.wait()

    def _pair(gg, _):
        for b in range(2):
            g = gg * 2 + b

            @pl.when(g + 1 < FULL // K)
            def _():
                _load(base + (g + 1) * K, 1 - b, K)

            _drain_load(b, K)

            @pl.when(gg >= 1)
            def _():
                pltpu.make_async_copy(
                    out_buf.at[b], wn_hbm.at[pl.ds(0, CE)], st[b]
                ).wait()

            _gather_rows(b, K)
            pltpu.async_copy(
                out_buf.at[b], wn_hbm.at[pl.ds((base + g * K) * LANES, CE)], st[b]
            )
        return 0

    lax.fori_loop(0, NPAIRS, _pair, 0)
    pltpu.make_async_copy(out_buf.at[0], wn_hbm.at[pl.ds(0, CE)], st[0]).wait()
    pltpu.make_async_copy(out_buf.at[1], wn_hbm.at[pl.ds(0, CE)], st[1]).wait()

    @pl.when(tid < 21)
    def _():
        _load(base + FULL, 0, EXTRA)
        _drain_load(0, EXTRA)
        _gather_rows(0, EXTRA)
        pltpu.sync_copy(
            out_buf.at[0, pl.ds(0, EXTRA * LANES)],
            wn_hbm.at[pl.ds((base + FULL) * LANES, EXTRA * LANES)],
        )

    @pl.when(tid >= 21)
    def _():
        _load(base + FULL, 1, TAIL)
        _drain_load(1, TAIL)
        _gather_rows(1, TAIL)
        pltpu.sync_copy(
            out_buf.at[1, pl.ds(0, TAIL * LANES)],
            wn_hbm.at[pl.ds((base + FULL) * LANES, TAIL * LANES)],
        )


def kernel(edge_index, weights):
    partial = _segment_sum_sc(edge_index, weights)
    inv_norm, row_sums = _finalize(partial)
    wn = _gather_mul_sc(edge_index, weights, inv_norm)
    return wn, row_sums.reshape(-1)[:N_ROWS]
